# baseline (speedup 1.0000x reference)
import jax
import jax.numpy as jnp
from jax import lax
from jax.experimental import pallas as pl
from jax.experimental.pallas import tpu as pltpu
from jax.experimental.pallas import tpu_sc as plsc

BATCH = 16384
NC = 2
NS = 16
NW = NC * NS
BPW = BATCH // NW
L = 16
CHUNK = 128
NCHUNK = BPW // CHUNK

USER_D = 32
GEO_D = 8
WIDE = 128
UPACK_C = 16384
PACK_BLK = 128

MLP_BB = 4096


def _user_pack_body(xin, pout):
    x = xin[...].astype(jnp.bfloat16)
    xt = jnp.transpose(x)
    q = UPACK_C // 8
    for k in range(8):
        pout[:, 32 * k:32 * (k + 1)] = xt[k * q:(k + 1) * q, :]


def _user_pack(utab_t):
    n = -(-utab_t.shape[1] // UPACK_C)
    return pl.pallas_call(
        _user_pack_body,
        grid=(n,),
        in_specs=[pl.BlockSpec((USER_D, UPACK_C), lambda i: (0, i))],
        out_specs=pl.BlockSpec((UPACK_C // 8, 2 * WIDE), lambda i: (i, 0)),
        out_shape=jax.ShapeDtypeStruct((n * UPACK_C // 8, 2 * WIDE), jnp.bfloat16),
        compiler_params=pltpu.CompilerParams(
            dimension_semantics=("arbitrary",)),
        name="user_pack",
    )(utab_t)


def _geo_pack_body(xin, pout):
    x = xin[...].astype(jnp.bfloat16)
    xt = jnp.transpose(x)
    q = UPACK_C // 32
    for k in range(32):
        pout[:, 8 * k:8 * (k + 1)] = xt[k * q:(k + 1) * q, :]


def _geo_pack(gtab_t):
    n = -(-gtab_t.shape[1] // UPACK_C)
    return pl.pallas_call(
        _geo_pack_body,
        grid=(n,),
        in_specs=[pl.BlockSpec((GEO_D, UPACK_C), lambda i: (0, i))],
        out_specs=pl.BlockSpec((UPACK_C // 32, 2 * WIDE), lambda i: (i, 0)),
        out_shape=jax.ShapeDtypeStruct((n * UPACK_C // 32, 2 * WIDE), jnp.bfloat16),
        compiler_params=pltpu.CompilerParams(
            dimension_semantics=("arbitrary",)),
        name="geo_pack",
    )(gtab_t)


def _sc_gather_body(uid_hbm, gcell_hbm, utab_hbm, gtab_hbm,
                    uout_hbm, gout_hbm,
                    uidx_v, gidx_v, rows_v, sem):
    c = lax.axis_index("c")
    s = lax.axis_index("s")
    wid = s * NC + c
    r0 = wid * NCHUNK
    b0 = wid * BPW

    pltpu.sync_copy(uid_hbm.at[pl.ds(r0, NCHUNK), :], uidx_v)
    pltpu.sync_copy(gcell_hbm.at[pl.ds(r0, NCHUNK), :], gidx_v)

    for j in range(NCHUNK):
        for i in range(CHUNK // L):
            sl = (j, pl.ds(i * L, L))
            u = uidx_v[sl]
            uidx_v[sl] = ((u >> 14) << 11) | (u & 2047)
            g = gidx_v[sl]
            gidx_v[sl] = ((g >> 14) << 9) | (g & 511)

    copies = []
    for j in range(NCHUNK):
        copies.append(pltpu.async_copy(
            utab_hbm.at[uidx_v.at[j]],
            rows_v.at[pl.ds(j * CHUNK, CHUNK)], sem))
    for cp in copies:
        cp.wait()
    pltpu.sync_copy(rows_v, uout_hbm.at[pl.ds(b0, BPW)])

    copies = []
    for j in range(NCHUNK):
        copies.append(pltpu.async_copy(
            gtab_hbm.at[gidx_v.at[j]],
            rows_v.at[pl.ds(j * CHUNK, CHUNK)], sem))
    for cp in copies:
        cp.wait()
    pltpu.sync_copy(rows_v, gout_hbm.at[pl.ds(b0, BPW)])


def _sc_gather(uid2d, gc2d, utab_packed, gtab_packed):
    mesh = plsc.VectorSubcoreMesh(
        core_axis_name="c", subcore_axis_name="s",
        num_cores=NC, num_subcores=NS)
    fn = pl.kernel(
        _sc_gather_body,
        out_type=(
            jax.ShapeDtypeStruct((BATCH, WIDE), jnp.uint32),
            jax.ShapeDtypeStruct((BATCH, WIDE), jnp.uint32),
        ),
        mesh=mesh,
        scratch_types=[
            pltpu.VMEM((NCHUNK, CHUNK), jnp.int32),
            pltpu.VMEM((NCHUNK, CHUNK), jnp.int32),
            pltpu.VMEM((BPW, WIDE), jnp.uint32),
            pltpu.SemaphoreType.DMA,
        ],
        name="sc_user_geo_gather",
    )
    return fn(uid2d, gc2d, utab_packed, gtab_packed)


def _mlp_body(uwide, gwide, ids, intr,
              atab, stab, w0, b0, w1, b1, w2, b2, out):
    f32 = jnp.float32
    hi = jax.lax.Precision.DEFAULT
    dn = (((1,), (0,)), ((), ()))

    ids_u = ids[:, 0:1]
    ids_g = ids[:, 1:2]

    lanes = lax.broadcasted_iota(jnp.int32, (MLP_BB, 2 * WIDE), 1)
    umask = (lanes >> 5 == ((ids_u >> 11) & 7)).astype(f32)
    gmask = (lanes >> 3 == ((ids_g >> 9) & 31)).astype(f32)

    qi = lax.broadcasted_iota(jnp.int32, (2 * WIDE, USER_D), 0)
    qj = lax.broadcasted_iota(jnp.int32, (2 * WIDE, USER_D), 1)
    qu = ((qi & 31) == qj).astype(f32)
    gi = lax.broadcasted_iota(jnp.int32, (2 * WIDE, GEO_D), 0)
    gj = lax.broadcasted_iota(jnp.int32, (2 * WIDE, GEO_D), 1)
    qg = ((gi & 7) == gj).astype(f32)

    u = lax.dot_general(uwide[...].astype(f32) * umask, qu, dn, precision=hi)
    geo = lax.dot_general(gwide[...].astype(f32) * gmask, qg, dn, precision=hi)

    ids_a = ids[:, 2:3]
    ids_s = ids[:, 3:4]
    iot = lax.broadcasted_iota(jnp.int32, (MLP_BB, 16), 1)
    aoh = (iot == ids_a).astype(f32)
    soh = (iot == ids_s).astype(f32)
    a_emb = lax.dot_general(aoh, atab[...], dn, precision=hi)
    s_emb = lax.dot_general(soh, stab[...], dn, precision=hi)

    x = jnp.concatenate([u, geo, a_emb, s_emb, intr[...]], axis=1)
    h = lax.dot_general(x, w0[...], dn, precision=hi) + b0[...]
    h = jnp.maximum(h, 0.0)
    h = lax.dot_general(h, w1[...], dn, precision=hi) + b1[...]
    h = jnp.maximum(h, 0.0)
    o = lax.dot_general(h, w2[...], dn, precision=hi) + b2[...]

    n2 = jnp.sum(o * o, axis=1, keepdims=True)
    out[...] = o * lax.rsqrt(jnp.maximum(n2, 1e-24))


def _mlp(uwide, gwide, ids4, interest,
         atab16, stab16, W0, b0, W1, b1, W2, b2):
    nblk = BATCH // MLP_BB
    bspec = lambda r, cols: pl.BlockSpec((r, cols), lambda i: (i, 0))
    full = lambda shape: pl.BlockSpec(shape, lambda i: (0, 0))
    return pl.pallas_call(
        _mlp_body,
        grid=(nblk,),
        in_specs=[
            bspec(MLP_BB, 2 * WIDE),
            bspec(MLP_BB, 2 * WIDE),
            bspec(MLP_BB, 4),
            bspec(MLP_BB, 64),
            full((16, 4)),
            full((16, 4)),
            full((112, 256)),
            full((1, 256)),
            full((256, 128)),
            full((1, 128)),
            full((128, 64)),
            full((1, 64)),
        ],
        out_specs=bspec(MLP_BB, 64),
        out_shape=jax.ShapeDtypeStruct((BATCH, 64), jnp.float32),
        compiler_params=pltpu.CompilerParams(
            dimension_semantics=("arbitrary",)),
        name="user_tower_mlp",
    )(uwide, gwide, ids4, interest,
      atab16, stab16, W0, b0, W1, b1, W2, b2)


def kernel(user_ids, geo_cells, age_buckets, schedule_types,
           interest_vectors, user_table, geo_table, age_table, sched_table,
           W0, b0, W1, b1, W2, b2):
    uid = user_ids.astype(jnp.int32)
    gc = geo_cells.astype(jnp.int32)
    ab = age_buckets.astype(jnp.int32)
    st = schedule_types.astype(jnp.int32)

    utab_packed = _user_pack(user_table.T)
    gtab_packed = _geo_pack(geo_table.T)
    as_u32 = lambda t: jax.lax.bitcast_convert_type(
        t.reshape(t.shape[0], WIDE, 2), jnp.uint32)
    as_bf = lambda t: jax.lax.bitcast_convert_type(
        t, jnp.bfloat16).reshape(-1, 2 * WIDE)

    u32, g32 = _sc_gather(uid.reshape(128, 128), gc.reshape(128, 128),
                          as_u32(utab_packed), as_u32(gtab_packed))
    uwide, gwide = as_bf(u32), as_bf(g32)

    atab16 = jnp.zeros((16, 4), jnp.float32).at[:age_table.shape[0]].set(age_table)
    stab16 = jnp.zeros((16, 4), jnp.float32).at[:sched_table.shape[0]].set(sched_table)

    ids4 = jnp.stack([uid, gc, ab, st], axis=1)
    return _mlp(uwide, gwide, ids4,
                interest_vectors,
                atab16, stab16,
                W0, b0.reshape(1, -1), W1, b1.reshape(1, -1),
                W2, b2.reshape(1, -1))


# speedup vs baseline: 2.8433x; 2.8433x over previous
import jax
import jax.numpy as jnp
from jax import lax
from jax.experimental import pallas as pl
from jax.experimental.pallas import tpu as pltpu
from jax.experimental.pallas import tpu_sc as plsc

BATCH = 16384
NC = 2
NS = 16
NW = NC * NS
BPW = BATCH // NW
L = 16
CHUNK = 128
NCHUNK = BPW // CHUNK

USER_D = 32
GEO_D = 8
WIDE = 128
UPACK_C = 16384
PACK_BLK = 128

MLP_BB = 4096


def _user_pack_body(xin, pout):
    x = xin[...]
    h = UPACK_C // 2
    bi = lax.broadcasted_iota(jnp.int32, (USER_D, USER_D), 0)
    bj = lax.broadcasted_iota(jnp.int32, (USER_D, USER_D), 1)
    eye = (bi == bj).astype(jnp.float32)
    xt1 = jnp.transpose(x[:, :h])
    xt2 = lax.dot_general(x[:, h:], eye, (((0,), (0,)), ((), ())))
    q = UPACK_C // 4
    pout[:, 0:32] = xt1[0:q, :]
    pout[:, 32:64] = xt1[q:2 * q, :]
    pout[:, 64:96] = xt2[0:q, :]
    pout[:, 96:128] = xt2[q:2 * q, :]


def _user_pack(utab_t):
    n = -(-utab_t.shape[1] // UPACK_C)
    return pl.pallas_call(
        _user_pack_body,
        grid=(n,),
        in_specs=[pl.BlockSpec((USER_D, UPACK_C), lambda i: (0, i))],
        out_specs=pl.BlockSpec((UPACK_C // 4, WIDE), lambda i: (i, 0)),
        out_shape=jax.ShapeDtypeStruct((n * UPACK_C // 4, WIDE), jnp.float32),
        compiler_params=pltpu.CompilerParams(
            dimension_semantics=("arbitrary",)),
        name="user_pack",
    )(utab_t)


def _geo_pack_body(xin, pout):
    x = xin[...]
    xt = jnp.transpose(x)
    q = UPACK_C // 16
    for k in range(16):
        pout[:, 8 * k:8 * k + 8] = xt[k * q:(k + 1) * q, :]


def _geo_pack(gtab_t):
    n = -(-gtab_t.shape[1] // UPACK_C)
    return pl.pallas_call(
        _geo_pack_body,
        grid=(n,),
        in_specs=[pl.BlockSpec((GEO_D, UPACK_C), lambda i: (0, i))],
        out_specs=pl.BlockSpec((UPACK_C // 16, WIDE), lambda i: (i, 0)),
        out_shape=jax.ShapeDtypeStruct((n * UPACK_C // 16, WIDE), jnp.float32),
        compiler_params=pltpu.CompilerParams(
            dimension_semantics=("arbitrary",)),
        name="geo_pack",
    )(gtab_t)


def _sc_gather_body(uid_hbm, gcell_hbm, utab_hbm, gtab_hbm,
                    uout_hbm, gout_hbm,
                    uidx_v, gidx_v, rows_v, sem):
    c = lax.axis_index("c")
    s = lax.axis_index("s")
    wid = s * NC + c
    r0 = wid * NCHUNK
    b0 = wid * BPW

    pltpu.sync_copy(uid_hbm.at[pl.ds(r0, NCHUNK), :], uidx_v)
    pltpu.sync_copy(gcell_hbm.at[pl.ds(r0, NCHUNK), :], gidx_v)

    for j in range(NCHUNK):
        for i in range(CHUNK // L):
            sl = (j, pl.ds(i * L, L))
            u = uidx_v[sl]
            uidx_v[sl] = ((u >> 14) << 12) | (u & 4095)
            g = gidx_v[sl]
            gidx_v[sl] = ((g >> 14) << 10) | (g & 1023)

    copies = []
    for j in range(NCHUNK):
        copies.append(pltpu.async_copy(
            utab_hbm.at[uidx_v.at[j]],
            rows_v.at[pl.ds(j * CHUNK, CHUNK)], sem))
    for cp in copies:
        cp.wait()
    pltpu.sync_copy(rows_v, uout_hbm.at[pl.ds(b0, BPW)])

    copies = []
    for j in range(NCHUNK):
        copies.append(pltpu.async_copy(
            gtab_hbm.at[gidx_v.at[j]],
            rows_v.at[pl.ds(j * CHUNK, CHUNK)], sem))
    for cp in copies:
        cp.wait()
    pltpu.sync_copy(rows_v, gout_hbm.at[pl.ds(b0, BPW)])


def _sc_gather(uid2d, gc2d, utab_packed, gtab_packed):
    mesh = plsc.VectorSubcoreMesh(
        core_axis_name="c", subcore_axis_name="s",
        num_cores=NC, num_subcores=NS)
    fn = pl.kernel(
        _sc_gather_body,
        out_type=(
            jax.ShapeDtypeStruct((BATCH, WIDE), jnp.float32),
            jax.ShapeDtypeStruct((BATCH, WIDE), jnp.float32),
        ),
        mesh=mesh,
        scratch_types=[
            pltpu.VMEM((NCHUNK, CHUNK), jnp.int32),
            pltpu.VMEM((NCHUNK, CHUNK), jnp.int32),
            pltpu.VMEM((BPW, WIDE), jnp.float32),
            pltpu.SemaphoreType.DMA,
        ],
        name="sc_user_geo_gather",
    )
    return fn(uid2d, gc2d, utab_packed, gtab_packed)


def _mlp_body(uwide, gwide, ids, intr,
              atab, stab, w0, b0, w1, b1, w2, b2, out):
    f32 = jnp.float32
    hi = jax.lax.Precision.DEFAULT
    dn = (((1,), (0,)), ((), ()))

    ids_u = ids[:, 0:1]
    ids_g = ids[:, 1:2]

    lanes = lax.broadcasted_iota(jnp.int32, (MLP_BB, WIDE), 1)
    umask = (lanes >> 5 == ((ids_u >> 12) & 3)).astype(f32)
    gmask = (lanes >> 3 == ((ids_g >> 10) & 15)).astype(f32)

    qi = lax.broadcasted_iota(jnp.int32, (WIDE, USER_D), 0)
    qj = lax.broadcasted_iota(jnp.int32, (WIDE, USER_D), 1)
    qu = ((qi & 31) == qj).astype(f32)
    gi = lax.broadcasted_iota(jnp.int32, (WIDE, GEO_D), 0)
    gj = lax.broadcasted_iota(jnp.int32, (WIDE, GEO_D), 1)
    qg = ((gi & 7) == gj).astype(f32)

    u = lax.dot_general(uwide[...] * umask, qu, dn, precision=hi)
    geo = lax.dot_general(gwide[...] * gmask, qg, dn, precision=hi)

    ids_a = ids[:, 2:3]
    ids_s = ids[:, 3:4]
    iot = lax.broadcasted_iota(jnp.int32, (MLP_BB, 16), 1)
    aoh = (iot == ids_a).astype(f32)
    soh = (iot == ids_s).astype(f32)
    a_emb = lax.dot_general(aoh, atab[...], dn, precision=hi)
    s_emb = lax.dot_general(soh, stab[...], dn, precision=hi)

    x = jnp.concatenate([u, geo, a_emb, s_emb, intr[...]], axis=1)
    h = lax.dot_general(x, w0[...], dn, precision=hi) + b0[...]
    h = jnp.maximum(h, 0.0)
    h = lax.dot_general(h, w1[...], dn, precision=hi) + b1[...]
    h = jnp.maximum(h, 0.0)
    o = lax.dot_general(h, w2[...], dn, precision=hi) + b2[...]

    n2 = jnp.sum(o * o, axis=1, keepdims=True)
    out[...] = o * lax.rsqrt(jnp.maximum(n2, 1e-24))


def _mlp(uwide, gwide, ids4, interest,
         atab16, stab16, W0, b0, W1, b1, W2, b2):
    nblk = BATCH // MLP_BB
    bspec = lambda r, cols: pl.BlockSpec((r, cols), lambda i: (i, 0))
    full = lambda shape: pl.BlockSpec(shape, lambda i: (0, 0))
    return pl.pallas_call(
        _mlp_body,
        grid=(nblk,),
        in_specs=[
            bspec(MLP_BB, WIDE),
            bspec(MLP_BB, WIDE),
            bspec(MLP_BB, 4),
            bspec(MLP_BB, 64),
            full((16, 4)),
            full((16, 4)),
            full((112, 256)),
            full((1, 256)),
            full((256, 128)),
            full((1, 128)),
            full((128, 64)),
            full((1, 64)),
        ],
        out_specs=bspec(MLP_BB, 64),
        out_shape=jax.ShapeDtypeStruct((BATCH, 64), jnp.float32),
        compiler_params=pltpu.CompilerParams(
            dimension_semantics=("arbitrary",)),
        name="user_tower_mlp",
    )(uwide, gwide, ids4, interest,
      atab16, stab16, W0, b0, W1, b1, W2, b2)


def kernel(user_ids, geo_cells, age_buckets, schedule_types,
           interest_vectors, user_table, geo_table, age_table, sched_table,
           W0, b0, W1, b1, W2, b2):
    uid = user_ids.astype(jnp.int32)
    gc = geo_cells.astype(jnp.int32)
    ab = age_buckets.astype(jnp.int32)
    st = schedule_types.astype(jnp.int32)

    utab_packed = _user_pack(user_table.T)
    gtab_packed = _geo_pack(geo_table.T)

    uwide, gwide = _sc_gather(uid.reshape(128, 128), gc.reshape(128, 128),
                              utab_packed, gtab_packed)

    atab16 = jnp.zeros((16, 4), jnp.float32).at[:age_table.shape[0]].set(age_table)
    stab16 = jnp.zeros((16, 4), jnp.float32).at[:sched_table.shape[0]].set(sched_table)

    ids4 = jnp.stack([uid, gc, ab, st], axis=1)
    return _mlp(uwide, gwide, ids4,
                interest_vectors,
                atab16, stab16,
                W0, b0.reshape(1, -1), W1, b1.reshape(1, -1),
                W2, b2.reshape(1, -1))
